# baseline (speedup 1.0000x reference)
import numpy as np
import jax
import jax.numpy as jnp
from jax import lax
from jax.experimental import pallas as pl
from jax.experimental.pallas import tpu as pltpu
from jax.experimental.pallas import tpu_sc as plsc

_PRIMES = (2654435761, 2246822519, 3266489917, 2028178513,
           1220703125, 1610612741, 805306457, 402653189)
_M = 2000000
_R32 = 967296
_R24 = 777216
_B, _T, _E = 1024, 200, 2
_TP = 224
_TK = 208
_NW = 32
_RPW = _B // _NW
_PPW = _RPW * _TK
_VPR = _TK // 16
_GC = 128
_NG = _PPW // _GC
_ORW = _TK * 8
_OCW = _T * 8

_Mi = np.int32(_M)


def _loop(n, body):
    def _step(i, _):
        body(i)
        return i + np.int32(1), None

    lax.scan(_step, np.int32(0), None, length=n)


def _round_bf16(v):
    b = lax.bitcast_convert_type(v, jnp.int32)
    r = b + np.int32(0x7FFF) + ((b >> np.int32(16)) & np.int32(1))
    return lax.bitcast_convert_type(r & np.int32(-65536), jnp.float32)


def _mod_buckets(x):
    q = (x.astype(jnp.float32) * jnp.float32(1.0 / _M)).astype(jnp.int32)
    r = x - q * _Mi
    r = jnp.where(r < 0, r + _Mi, r)
    r = jnp.where(r >= _Mi, r - _Mi, r)
    return r


def _digits_mod(l0, l1, l2):
    x = (l2 * np.int32(_R32) + (l1 >> np.int32(8)) * np.int32(_R24)
         + (l1 & np.int32(255)) * np.int32(65536) + l0)
    return _mod_buckets(x)


def _body(tok_hbm, w00, w01, w10, w11, w20, w21, w30, w31, cw_hbm, out_hbm,
          tok_v, key0, key1, key2, key3, dsta, dstb, cw_v, out_v, sem):
    wid = lax.axis_index("s") * np.int32(2) + lax.axis_index("c")
    pltpu.sync_copy(tok_hbm.at[pl.ds(wid * np.int32(_RPW * _TP), _RPW * _TP)],
                    tok_v)
    pltpu.sync_copy(cw_hbm, cw_v)

    def round_cw(i):
        o = i * np.int32(16)
        cw_v[pl.ds(o, 16)] = _round_bf16(cw_v[pl.ds(o, 16)])
    _loop(32, round_cw)

    iota = lax.iota(jnp.int32, 16)
    zero16 = jnp.zeros((16,), jnp.int32)

    def hash_row(r):
        def hash_vec(jv):
            p0 = r * np.int32(_TK) + jv * np.int32(16)
            tbase = r * np.int32(_TP) + jv * np.int32(16) + np.int32(8)
            l0, l1, l2 = zero16, zero16, zero16
            for i in range(8):
                p = _PRIMES[i]
                t = tok_v[pl.ds(tbase - np.int32(i + 1), 16)]
                a = t * np.int32(p >> 16)
                b = t * np.int32(p & 0xFFFF)
                l0 = l0 ^ (b & np.int32(0xFFFF))
                m = (b >> np.int32(16)) + (a & np.int32(0xFFFF))
                l1 = l1 ^ (m & np.int32(0xFFFF))
                l2 = l2 ^ ((a >> np.int32(16)) + (m >> np.int32(16)))
                if i == 0:
                    key0[pl.ds(p0, 16)] = _digits_mod(l0, l1, l2)
                elif i == 1:
                    key1[pl.ds(p0, 16)] = _digits_mod(l0, l1, l2)
                elif i == 3:
                    key2[pl.ds(p0, 16)] = _digits_mod(l0, l1, l2)
                elif i == 7:
                    key3[pl.ds(p0, 16)] = _digits_mod(l0, l1, l2)
        _loop(_VPR, hash_vec)
    _loop(_RPW, hash_row)

    def fire2(kref, c0, c1, dst, o):
        pltpu.async_copy(c0.at[kref.at[pl.ds(o, _GC)]],
                         dst.at[pl.ds(o, _GC)], sem)
        pltpu.async_copy(c1.at[kref.at[pl.ds(o, _GC)]],
                         dst.at[pl.ds(np.int32(_PPW) + o, _GC)], sem)

    def drain2(kref, c0, c1, dst, o):
        pltpu.make_async_copy(c0.at[kref.at[pl.ds(o, _GC)]],
                              dst.at[pl.ds(o, _GC)], sem).wait()
        pltpu.make_async_copy(c1.at[kref.at[pl.ds(o, _GC)]],
                              dst.at[pl.ds(np.int32(_PPW) + o, _GC)],
                              sem).wait()

    def fire_short(g):
        o = g * np.int32(_GC)
        fire2(key0, w00, w01, dsta, o)
        fire2(key1, w10, w11, dstb, o)
    _loop(_NG, fire_short)

    def drain_short(g):
        o = g * np.int32(_GC)
        drain2(key0, w00, w01, dsta, o)
        drain2(key1, w10, w11, dstb, o)
    _loop(_NG, drain_short)

    def cond_row(r):
        def cond_vec(jv):
            p0 = r * np.int32(_TK) + jv * np.int32(16)
            posv = p0 + iota
            e = []
            for dst in (dsta, dstb):
                for c in (0, 1):
                    e.append(plsc.load_gather(
                        dst, [posv + np.int32(c * _PPW)]))
            eb = [_round_bf16(v) for v in e]
            ck0, ck1 = zero16, zero16
            for j in range(8):
                lg = (eb[0] * cw_v[pl.ds(4 * j * 16, 16)]
                      + eb[1] * cw_v[pl.ds((4 * j + 1) * 16, 16)]
                      + eb[2] * cw_v[pl.ds((4 * j + 2) * 16, 16)]
                      + eb[3] * cw_v[pl.ds((4 * j + 3) * 16, 16)])
                sb = (lg > jnp.float32(0.0)).astype(jnp.int32)
                ck0 = ck0 ^ (sb * np.int32(_PRIMES[j] & 0xFFFF))
                ck1 = ck1 ^ (sb * np.int32(_PRIMES[j] >> 16))
            for key in (key2, key3):
                k = key[pl.ds(p0, 16)]
                x0 = (k & np.int32(0xFFFF)) ^ ck0
                x1 = (k >> np.int32(16)) ^ ck1
                x = ((x1 >> np.int32(8)) * np.int32(_R24)
                     + (x1 & np.int32(255)) * np.int32(65536) + x0)
                key[pl.ds(p0, 16)] = _mod_buckets(x)
            obase = posv * np.int32(8)
            for c in (0, 1):
                plsc.store_scatter(out_v, [obase + np.int32(c)], e[c])
                plsc.store_scatter(out_v, [obase + np.int32(2 + c)], e[2 + c])
        _loop(_VPR, cond_vec)
    _loop(_RPW, cond_row)

    def fire_long(g):
        o = g * np.int32(_GC)
        fire2(key2, w20, w21, dsta, o)
        fire2(key3, w30, w31, dstb, o)
    _loop(_NG, fire_long)

    def drain_long(g):
        o = g * np.int32(_GC)
        drain2(key2, w20, w21, dsta, o)
        drain2(key3, w30, w31, dstb, o)
    _loop(_NG, drain_long)

    def emit_row(r):
        def emit_vec(jv):
            p0 = r * np.int32(_TK) + jv * np.int32(16)
            posv = p0 + iota
            obase = posv * np.int32(8)
            for base, dst in ((4, dsta), (6, dstb)):
                for c in (0, 1):
                    v = plsc.load_gather(dst, [posv + np.int32(c * _PPW)])
                    plsc.store_scatter(out_v, [obase + np.int32(base + c)], v)
        _loop(_VPR, emit_vec)
    _loop(_RPW, emit_row)

    def out_row(r):
        pltpu.sync_copy(
            out_v.at[pl.ds(r * np.int32(_ORW), _OCW)],
            out_hbm.at[pl.ds((wid * np.int32(_RPW) + r) * np.int32(_OCW),
                             _OCW)])
    _loop(_RPW, out_row)


@jax.jit
def _pyramid_sc(tok_flat, w00, w01, w10, w11, w20, w21, w30, w31, cwb):
    mesh = plsc.VectorSubcoreMesh(core_axis_name="c", subcore_axis_name="s")
    call = pl.kernel(
        _body,
        out_type=jax.ShapeDtypeStruct((_B * _T * 8,), jnp.float32),
        mesh=mesh,
        compiler_params=pltpu.CompilerParams(needs_layout_passes=False),
        scratch_types=[
            pltpu.VMEM((_RPW * _TP,), jnp.int32),
            pltpu.VMEM((_PPW,), jnp.int32),
            pltpu.VMEM((_PPW,), jnp.int32),
            pltpu.VMEM((_PPW,), jnp.int32),
            pltpu.VMEM((_PPW,), jnp.int32),
            pltpu.VMEM((2 * _PPW,), jnp.float32),
            pltpu.VMEM((2 * _PPW,), jnp.float32),
            pltpu.VMEM((512,), jnp.float32),
            pltpu.VMEM((_RPW * _TK * 8,), jnp.float32),
            pltpu.SemaphoreType.DMA,
        ],
    )
    return call(tok_flat, w00, w01, w10, w11, w20, w21, w30, w31, cwb)


def kernel(tokens, W0, W1, W2, W3, cond_W):
    zf = lax.optimization_barrier(jnp.zeros((1,), jnp.float32))
    zi = lax.optimization_barrier(jnp.zeros((1,), jnp.int32))
    tok = tokens.astype(jnp.int32)
    tok_flat = jnp.pad(tok, ((0, 0), (8, _TP - _T - 8))).reshape(-1) + zi
    cwb = jnp.repeat(cond_W.astype(jnp.float32).reshape(32), 16)
    cols = []
    for W in (W0, W1, W2, W3):
        cols.append(W[:, 0])
        cols.append(W[:, 1])
    out = _pyramid_sc(tok_flat, *cols, cwb)
    return out.reshape(_B, _T, 8) + zf


# speedup vs baseline: 1.1505x; 1.1505x over previous
import numpy as np
import jax
import jax.numpy as jnp
from jax import lax
from jax.experimental import pallas as pl
from jax.experimental.pallas import tpu as pltpu
from jax.experimental.pallas import tpu_sc as plsc

_PRIMES = (2654435761, 2246822519, 3266489917, 2028178513,
           1220703125, 1610612741, 805306457, 402653189)
_M = 2000000
_R32 = 967296
_R24 = 777216
_B, _T, _E = 1024, 200, 2
_TP = 224
_TK = 208
_NW = 32
_RPW = _B // _NW
_PPW = _RPW * _TK
_VPR = _TK // 16
_GC = 104
_NG = _PPW // _GC
_ORW = _TK * 8
_OCW = _T * 8

_Mi = np.int32(_M)


def _loop(n, body):
    def _step(i, _):
        body(i)
        return i + np.int32(1), None

    lax.scan(_step, np.int32(0), None, length=n)


def _round_bf16(v):
    b = lax.bitcast_convert_type(v, jnp.int32)
    r = b + np.int32(0x7FFF) + ((b >> np.int32(16)) & np.int32(1))
    return lax.bitcast_convert_type(r & np.int32(-65536), jnp.float32)


def _mod_buckets(x):
    q = (x.astype(jnp.float32) * jnp.float32(1.0 / _M)).astype(jnp.int32)
    r = x - q * _Mi
    r = jnp.where(r < 0, r + _Mi, r)
    r = jnp.where(r >= _Mi, r - _Mi, r)
    return r


def _digits_mod(l0, l1, l2):
    x = (l2 * np.int32(_R32) + (l1 >> np.int32(8)) * np.int32(_R24)
         + (l1 & np.int32(255)) * np.int32(65536) + l0)
    return _mod_buckets(x)


def _body(tok_hbm, w00, w01, w10, w11, w20, w21, w30, w31, cw_hbm, out_hbm,
          tok_v, key0, key1, key2, key3, dsta, dstb, cw_v, out_v, sem):
    wid = lax.axis_index("s") * np.int32(2) + lax.axis_index("c")
    pltpu.sync_copy(tok_hbm.at[pl.ds(wid * np.int32(_RPW * _TP), _RPW * _TP)],
                    tok_v)
    pltpu.sync_copy(cw_hbm, cw_v)

    def round_cw(i):
        o = i * np.int32(16)
        cw_v[pl.ds(o, 16)] = _round_bf16(cw_v[pl.ds(o, 16)])
    _loop(32, round_cw)

    iota = lax.iota(jnp.int32, 16)
    zero16 = jnp.zeros((16,), jnp.int32)

    def fire2(kref, c0, c1, dst, o):
        pltpu.async_copy(c0.at[kref.at[pl.ds(o, _GC)]],
                         dst.at[pl.ds(o, _GC)], sem)
        pltpu.async_copy(c1.at[kref.at[pl.ds(o, _GC)]],
                         dst.at[pl.ds(np.int32(_PPW) + o, _GC)], sem)

    def drain2(kref, c0, c1, dst, o):
        pltpu.make_async_copy(c0.at[kref.at[pl.ds(o, _GC)]],
                              dst.at[pl.ds(o, _GC)], sem).wait()
        pltpu.make_async_copy(c1.at[kref.at[pl.ds(o, _GC)]],
                              dst.at[pl.ds(np.int32(_PPW) + o, _GC)],
                              sem).wait()

    def hash_row(r):
        def hash_vec(jv):
            p0 = r * np.int32(_TK) + jv * np.int32(16)
            tbase = r * np.int32(_TP) + jv * np.int32(16) + np.int32(8)
            l0, l1, l2 = zero16, zero16, zero16
            for i in range(8):
                p = _PRIMES[i]
                t = tok_v[pl.ds(tbase - np.int32(i + 1), 16)]
                a = t * np.int32(p >> 16)
                b = t * np.int32(p & 0xFFFF)
                l0 = l0 ^ (b & np.int32(0xFFFF))
                m = (b >> np.int32(16)) + (a & np.int32(0xFFFF))
                l1 = l1 ^ (m & np.int32(0xFFFF))
                l2 = l2 ^ ((a >> np.int32(16)) + (m >> np.int32(16)))
                if i == 0:
                    key0[pl.ds(p0, 16)] = _digits_mod(l0, l1, l2)
                elif i == 1:
                    key1[pl.ds(p0, 16)] = _digits_mod(l0, l1, l2)
                elif i == 3:
                    key2[pl.ds(p0, 16)] = _digits_mod(l0, l1, l2)
                elif i == 7:
                    key3[pl.ds(p0, 16)] = _digits_mod(l0, l1, l2)
        _loop(_VPR, hash_vec)
        for half in (0, 1):
            o = r * np.int32(_TK) + np.int32(half * _GC)
            fire2(key0, w00, w01, dsta, o)
            fire2(key1, w10, w11, dstb, o)
    _loop(_RPW, hash_row)

    def drain_short(g):
        o = g * np.int32(_GC)
        drain2(key0, w00, w01, dsta, o)
        drain2(key1, w10, w11, dstb, o)
    _loop(_NG, drain_short)

    def cond_row(r):
        def cond_vec(jv):
            p0 = r * np.int32(_TK) + jv * np.int32(16)
            posv = p0 + iota
            e = []
            for dst in (dsta, dstb):
                for c in (0, 1):
                    e.append(plsc.load_gather(
                        dst, [posv + np.int32(c * _PPW)]))
            eb = [_round_bf16(v) for v in e]
            ck0, ck1 = zero16, zero16
            for j in range(8):
                lg = (eb[0] * cw_v[pl.ds(4 * j * 16, 16)]
                      + eb[1] * cw_v[pl.ds((4 * j + 1) * 16, 16)]
                      + eb[2] * cw_v[pl.ds((4 * j + 2) * 16, 16)]
                      + eb[3] * cw_v[pl.ds((4 * j + 3) * 16, 16)])
                sb = (lg > jnp.float32(0.0)).astype(jnp.int32)
                ck0 = ck0 ^ (sb * np.int32(_PRIMES[j] & 0xFFFF))
                ck1 = ck1 ^ (sb * np.int32(_PRIMES[j] >> 16))
            for key in (key2, key3):
                k = key[pl.ds(p0, 16)]
                x0 = (k & np.int32(0xFFFF)) ^ ck0
                x1 = (k >> np.int32(16)) ^ ck1
                x = ((x1 >> np.int32(8)) * np.int32(_R24)
                     + (x1 & np.int32(255)) * np.int32(65536) + x0)
                key[pl.ds(p0, 16)] = _mod_buckets(x)
            obase = posv * np.int32(8)
            for c in (0, 1):
                plsc.store_scatter(out_v, [obase + np.int32(c)], e[c])
                plsc.store_scatter(out_v, [obase + np.int32(2 + c)], e[2 + c])
        _loop(_VPR, cond_vec)
        for half in (0, 1):
            o = r * np.int32(_TK) + np.int32(half * _GC)
            fire2(key2, w20, w21, dsta, o)
            fire2(key3, w30, w31, dstb, o)
    _loop(_RPW, cond_row)

    def drain_long(g):
        o = g * np.int32(_GC)
        drain2(key2, w20, w21, dsta, o)
        drain2(key3, w30, w31, dstb, o)
    _loop(_NG, drain_long)

    def emit_row(r):
        def emit_vec(jv):
            p0 = r * np.int32(_TK) + jv * np.int32(16)
            posv = p0 + iota
            obase = posv * np.int32(8)
            for base, dst in ((4, dsta), (6, dstb)):
                for c in (0, 1):
                    v = plsc.load_gather(dst, [posv + np.int32(c * _PPW)])
                    plsc.store_scatter(out_v, [obase + np.int32(base + c)], v)
        _loop(_VPR, emit_vec)
        pltpu.async_copy(
            out_v.at[pl.ds(r * np.int32(_ORW), _OCW)],
            out_hbm.at[pl.ds((wid * np.int32(_RPW) + r) * np.int32(_OCW),
                             _OCW)], sem)
    _loop(_RPW, emit_row)

    def out_drain(r):
        pltpu.make_async_copy(
            out_v.at[pl.ds(r * np.int32(_ORW), _OCW)],
            out_hbm.at[pl.ds((wid * np.int32(_RPW) + r) * np.int32(_OCW),
                             _OCW)], sem).wait()
    _loop(_RPW, out_drain)


@jax.jit
def _pyramid_sc(tok_flat, w00, w01, w10, w11, w20, w21, w30, w31, cwb):
    mesh = plsc.VectorSubcoreMesh(core_axis_name="c", subcore_axis_name="s")
    call = pl.kernel(
        _body,
        out_type=jax.ShapeDtypeStruct((_B * _T * 8,), jnp.float32),
        mesh=mesh,
        compiler_params=pltpu.CompilerParams(needs_layout_passes=False),
        scratch_types=[
            pltpu.VMEM((_RPW * _TP,), jnp.int32),
            pltpu.VMEM((_PPW,), jnp.int32),
            pltpu.VMEM((_PPW,), jnp.int32),
            pltpu.VMEM((_PPW,), jnp.int32),
            pltpu.VMEM((_PPW,), jnp.int32),
            pltpu.VMEM((2 * _PPW,), jnp.float32),
            pltpu.VMEM((2 * _PPW,), jnp.float32),
            pltpu.VMEM((512,), jnp.float32),
            pltpu.VMEM((_RPW * _TK * 8,), jnp.float32),
            pltpu.SemaphoreType.DMA,
        ],
    )
    return call(tok_flat, w00, w01, w10, w11, w20, w21, w30, w31, cwb)


def kernel(tokens, W0, W1, W2, W3, cond_W):
    tok = tokens.astype(jnp.int32)
    tok_flat = jnp.pad(tok, ((0, 0), (8, _TP - _T - 8))).reshape(-1)
    cwb = jnp.repeat(cond_W.astype(jnp.float32).reshape(32), 16)
    cols = []
    for W in (W0, W1, W2, W3):
        cols.append(W[:, 0])
        cols.append(W[:, 1])
    return _pyramid_sc(tok_flat, *cols, cwb).reshape(_B, _T, 8)


# speedup vs baseline: 1.1536x; 1.0027x over previous
import numpy as np
import jax
import jax.numpy as jnp
from jax import lax
from jax.experimental import pallas as pl
from jax.experimental.pallas import tpu as pltpu
from jax.experimental.pallas import tpu_sc as plsc

_PRIMES = (2654435761, 2246822519, 3266489917, 2028178513,
           1220703125, 1610612741, 805306457, 402653189)
_M = 2000000
_R32 = 967296
_R24 = 777216
_B, _T, _E = 1024, 200, 2
_TP = 224
_TK = 208
_NW = 32
_RPW = _B // _NW
_PPW = _RPW * _TK
_VPR = _TK // 16
_GC = 104
_NG = _PPW // _GC
_ORW = _TK * 8
_OCW = _T * 8

_Mi = np.int32(_M)


def _loop(n, body):
    def _step(i, _):
        body(i)
        return i + np.int32(1), None

    lax.scan(_step, np.int32(0), None, length=n)


def _round_bf16(v):
    b = lax.bitcast_convert_type(v, jnp.int32)
    r = b + np.int32(0x7FFF) + ((b >> np.int32(16)) & np.int32(1))
    return lax.bitcast_convert_type(r & np.int32(-65536), jnp.float32)


def _mod_buckets(x):
    q = (x.astype(jnp.float32) * jnp.float32(1.0 / _M)).astype(jnp.int32)
    r = x - q * _Mi
    r = jnp.where(r < 0, r + _Mi, r)
    r = jnp.where(r >= _Mi, r - _Mi, r)
    return r


def _digits_mod(l0, l1, l2):
    x = (l2 * np.int32(_R32) + (l1 >> np.int32(8)) * np.int32(_R24)
         + (l1 & np.int32(255)) * np.int32(65536) + l0)
    return _mod_buckets(x)


def _body(tok_hbm, w00, w01, w10, w11, w20, w21, w30, w31, cw_hbm, out_hbm,
          tok_v, key0, key1, key2, key3, dsta, dstb, cw_v, out_v, sem):
    wid = lax.axis_index("s") * np.int32(2) + lax.axis_index("c")
    pltpu.sync_copy(tok_hbm.at[pl.ds(wid * np.int32(_RPW * _TP), _RPW * _TP)],
                    tok_v)
    pltpu.sync_copy(cw_hbm, cw_v)

    def round_cw(i):
        o = i * np.int32(16)
        cw_v[pl.ds(o, 16)] = _round_bf16(cw_v[pl.ds(o, 16)])
    _loop(32, round_cw)

    iota = lax.iota(jnp.int32, 16)
    zero16 = jnp.zeros((16,), jnp.int32)

    def fire2(kref, c0, c1, dst, o):
        pltpu.async_copy(c0.at[kref.at[pl.ds(o, _GC)]],
                         dst.at[pl.ds(o, _GC)], sem)
        pltpu.async_copy(c1.at[kref.at[pl.ds(o, _GC)]],
                         dst.at[pl.ds(np.int32(_PPW) + o, _GC)], sem)

    def drain2(kref, c0, c1, dst, o):
        pltpu.make_async_copy(c0.at[kref.at[pl.ds(o, _GC)]],
                              dst.at[pl.ds(o, _GC)], sem).wait()
        pltpu.make_async_copy(c1.at[kref.at[pl.ds(o, _GC)]],
                              dst.at[pl.ds(np.int32(_PPW) + o, _GC)],
                              sem).wait()

    def hash_row(r):
        def hash_vec(jv):
            p0 = r * np.int32(_TK) + jv * np.int32(16)
            tbase = r * np.int32(_TP) + jv * np.int32(16) + np.int32(8)
            l0, l1, l2 = zero16, zero16, zero16
            for i in range(8):
                p = _PRIMES[i]
                t = tok_v[pl.ds(tbase - np.int32(i + 1), 16)]
                a = t * np.int32(p >> 16)
                b = t * np.int32(p & 0xFFFF)
                l0 = l0 ^ (b & np.int32(0xFFFF))
                m = (b >> np.int32(16)) + (a & np.int32(0xFFFF))
                l1 = l1 ^ (m & np.int32(0xFFFF))
                l2 = l2 ^ ((a >> np.int32(16)) + (m >> np.int32(16)))
                if i == 0:
                    key0[pl.ds(p0, 16)] = _digits_mod(l0, l1, l2)
                elif i == 1:
                    key1[pl.ds(p0, 16)] = _digits_mod(l0, l1, l2)
                elif i == 3:
                    key2[pl.ds(p0, 16)] = _digits_mod(l0, l1, l2)
                elif i == 7:
                    key3[pl.ds(p0, 16)] = _digits_mod(l0, l1, l2)
        _loop(_VPR, hash_vec)
        for half in (0, 1):
            o = r * np.int32(_TK) + np.int32(half * _GC)
            fire2(key0, w00, w01, dsta, o)
            fire2(key1, w10, w11, dstb, o)
    _loop(_RPW, hash_row)

    def drain_short(g):
        o = g * np.int32(_GC)
        drain2(key0, w00, w01, dsta, o)
        drain2(key1, w10, w11, dstb, o)
    _loop(_NG, drain_short)

    def cond_row(r):
        def cond_vec(jv):
            p0 = r * np.int32(_TK) + jv * np.int32(16)
            posv = p0 + iota
            e = []
            for dst in (dsta, dstb):
                for c in (0, 1):
                    e.append(plsc.load_gather(
                        dst, [posv + np.int32(c * _PPW)]))
            eb = [_round_bf16(v) for v in e]
            ck0, ck1 = zero16, zero16
            for j in range(8):
                lg = (eb[0] * cw_v[pl.ds(4 * j * 16, 16)]
                      + eb[1] * cw_v[pl.ds((4 * j + 1) * 16, 16)]
                      + eb[2] * cw_v[pl.ds((4 * j + 2) * 16, 16)]
                      + eb[3] * cw_v[pl.ds((4 * j + 3) * 16, 16)])
                sb = (lg > jnp.float32(0.0)).astype(jnp.int32)
                ck0 = ck0 ^ (sb * np.int32(_PRIMES[j] & 0xFFFF))
                ck1 = ck1 ^ (sb * np.int32(_PRIMES[j] >> 16))
            for key in (key2, key3):
                k = key[pl.ds(p0, 16)]
                x0 = (k & np.int32(0xFFFF)) ^ ck0
                x1 = (k >> np.int32(16)) ^ ck1
                x = ((x1 >> np.int32(8)) * np.int32(_R24)
                     + (x1 & np.int32(255)) * np.int32(65536) + x0)
                key[pl.ds(p0, 16)] = _mod_buckets(x)
            obase = posv * np.int32(8)
            for c in (0, 1):
                plsc.store_scatter(out_v, [obase + np.int32(c)], e[c])
                plsc.store_scatter(out_v, [obase + np.int32(2 + c)], e[2 + c])
        _loop(_VPR, cond_vec)
        for half in (0, 1):
            o = r * np.int32(_TK) + np.int32(half * _GC)
            fire2(key2, w20, w21, dsta, o)
            fire2(key3, w30, w31, dstb, o)
    _loop(_RPW, cond_row)

    def drain_long(g):
        o = g * np.int32(_GC)
        drain2(key2, w20, w21, dsta, o)
        drain2(key3, w30, w31, dstb, o)
    _loop(_NG, drain_long)

    def emit_row(r):
        def emit_vec(jv):
            p0 = r * np.int32(_TK) + jv * np.int32(16)
            posv = p0 + iota
            obase = posv * np.int32(8)
            for base, dst in ((4, dsta), (6, dstb)):
                for c in (0, 1):
                    v = plsc.load_gather(dst, [posv + np.int32(c * _PPW)])
                    plsc.store_scatter(out_v, [obase + np.int32(base + c)], v)
        _loop(_VPR, emit_vec)
        pltpu.async_copy(
            out_v.at[pl.ds(r * np.int32(_ORW), _OCW)],
            out_hbm.at[pl.ds((wid * np.int32(_RPW) + r) * np.int32(_OCW),
                             _OCW)], sem)
    _loop(_RPW, emit_row)

    def out_drain(r):
        pltpu.make_async_copy(
            out_v.at[pl.ds(r * np.int32(_ORW), _OCW)],
            out_hbm.at[pl.ds((wid * np.int32(_RPW) + r) * np.int32(_OCW),
                             _OCW)], sem).wait()
    _loop(_RPW, out_drain)


@jax.jit
def _pyramid_sc(tok_flat, w00, w01, w10, w11, w20, w21, w30, w31, cwb):
    mesh = plsc.VectorSubcoreMesh(core_axis_name="c", subcore_axis_name="s")
    call = pl.kernel(
        _body,
        out_type=jax.ShapeDtypeStruct((_B * _T * 8,), jnp.float32),
        mesh=mesh,
        compiler_params=pltpu.CompilerParams(needs_layout_passes=False),
        scratch_types=[
            pltpu.VMEM((_RPW * _TP,), jnp.int32),
            pltpu.VMEM((_PPW,), jnp.int32),
            pltpu.VMEM((_PPW,), jnp.int32),
            pltpu.VMEM((_PPW,), jnp.int32),
            pltpu.VMEM((_PPW,), jnp.int32),
            pltpu.VMEM((2 * _PPW,), jnp.float32),
            pltpu.VMEM((2 * _PPW,), jnp.float32),
            pltpu.VMEM((512,), jnp.float32),
            pltpu.VMEM((_RPW * _TK * 8,), jnp.float32),
            pltpu.SemaphoreType.DMA,
        ],
    )
    return call(tok_flat, w00, w01, w10, w11, w20, w21, w30, w31, cwb)


def kernel(tokens, W0, W1, W2, W3, cond_W):
    zi = lax.optimization_barrier(jnp.zeros((1,), jnp.int32))
    tok = tokens.astype(jnp.int32)
    tok_flat = jnp.pad(tok, ((0, 0), (8, _TP - _T - 8))).reshape(-1) + zi
    cwb = jnp.repeat(cond_W.astype(jnp.float32).reshape(32), 16)
    cols = []
    for W in (W0, W1, W2, W3):
        cols.append(W[:, 0])
        cols.append(W[:, 1])
    return _pyramid_sc(tok_flat, *cols, cwb).reshape(_B, _T, 8)
